# baseline (speedup 1.0000x reference)
import jax
import jax.numpy as jnp
from jax import lax
from jax.experimental import pallas as pl
from jax.experimental.pallas import tpu as pltpu
from jax.experimental.pallas import tpu_sc as plsc

N = 10000
E = 320000
EH = E // 2
ND = 128
DE = 16

NC = 2
NS = 16
NW = NC * NS
CH = E // NW
CHH = EH // NW
GC = 200
NGH = CHH // GC
TCH = EH // NS
DC = 160
NDC = 62
DCT = TCH - NDC * DC
ZR = 4
TOT = NS * ZR * DC
WCH = 624
NEG = -1e30

BA = 1000
BE = 2048
BN = 1000


def _ln_relu(h, g, b):
    mu = jnp.mean(h, axis=-1, keepdims=True)
    d = h - mu
    var = jnp.mean(d * d, axis=-1, keepdims=True)
    return jnp.maximum(d * lax.rsqrt(var + 1e-5) * g + b, 0.0)


def _prep_body(x_ref, wa_ref, wb_ref, ab_ref, ws_ref, bs_ref, gs_ref, bes_ref,
               xn_ref, pc_ref, pr_ref, s_ref):
    xn = x_ref[:, :ND]
    xn_ref[...] = xn
    pc_ref[...] = jnp.dot(xn, wa_ref[...], preferred_element_type=jnp.float32) + ab_ref[0]
    pr_ref[...] = jnp.dot(xn, wb_ref[...], preferred_element_type=jnp.float32)
    h = jnp.dot(xn, ws_ref[...], preferred_element_type=jnp.float32) + bs_ref[...]
    s_ref[...] = _ln_relu(h, gs_ref[...], bes_ref[...])


_prep = pl.pallas_call(
    _prep_body,
    grid=(N // BA,),
    in_specs=[
        pl.BlockSpec((BA, ND + DE), lambda i: (i, 0)),
        pl.BlockSpec((ND, 1), lambda i: (0, 0)),
        pl.BlockSpec((ND, 1), lambda i: (0, 0)),
        pl.BlockSpec(memory_space=pltpu.SMEM),
        pl.BlockSpec((ND, ND), lambda i: (0, 0)),
        pl.BlockSpec((1, ND), lambda i: (0, 0)),
        pl.BlockSpec((1, ND), lambda i: (0, 0)),
        pl.BlockSpec((1, ND), lambda i: (0, 0)),
    ],
    out_specs=[
        pl.BlockSpec((BA, ND), lambda i: (i, 0)),
        pl.BlockSpec((BA, 1), lambda i: (i, 0)),
        pl.BlockSpec((BA, 1), lambda i: (i, 0)),
        pl.BlockSpec((BA, ND), lambda i: (i, 0)),
    ],
    out_shape=[
        jax.ShapeDtypeStruct((N, ND), jnp.float32),
        jax.ShapeDtypeStruct((N, 1), jnp.float32),
        jax.ShapeDtypeStruct((N, 1), jnp.float32),
        jax.ShapeDtypeStruct((N, ND), jnp.float32),
    ],
)


def _logits_body(row_hbm, col_hbm, pc_hbm, pr_hbm,
                 l1_out, l2_out, idx_out, stats_out,
                 pc_v, pr_v, r_v, c_v, l1_v, l2_v, idx_v, stat_v):
    wid = lax.axis_index("s") * NC + lax.axis_index("c")
    base = wid * CH
    pltpu.sync_copy(pc_hbm, pc_v)
    pltpu.sync_copy(pr_hbm, pr_v)
    pltpu.sync_copy(row_hbm.at[pl.ds(base, CH)], r_v)
    pltpu.sync_copy(col_hbm.at[pl.ds(base, CH)], c_v)

    negv = jnp.full((16,), NEG, jnp.float32)

    def body(i, carry):
        m1v, m2v = carry
        for k in range(4):
            o = pl.multiple_of(i * 64 + k * 16, 16)
            r = r_v[pl.ds(o, 16)]
            c = c_v[pl.ds(o, 16)]
            lc = plsc.load_gather(pc_v, [c])
            lr = plsc.load_gather(pr_v, [r])
            s = lc + lr
            l = jnp.where(s >= 0.0, s, s * 0.01)
            m1 = r < c
            m2 = r > c
            l1 = jnp.where(m1, l, NEG)
            l2 = jnp.where(m2, l, NEG)
            l1_v[pl.ds(o, 16)] = l1
            l2_v[pl.ds(o, 16)] = l2
            idx_v[pl.ds(o, 16)] = jnp.where(m1, r + N, jnp.where(m2, r, 2 * N))
            m1v = jnp.maximum(m1v, l1)
            m2v = jnp.maximum(m2v, l2)
        return (m1v, m2v)

    m1v, m2v = lax.fori_loop(0, CH // 64, body, (negv, negv))
    m1w = jnp.max(m1v)
    m2w = jnp.max(m2v)
    zv16 = jnp.zeros((16,), jnp.float32)

    def sbody(i, carry):
        s1v, s2v = carry
        for k in range(4):
            o = pl.multiple_of(i * 64 + k * 16, 16)
            s1v = s1v + jnp.exp(l1_v[pl.ds(o, 16)] - m1w)
            s2v = s2v + jnp.exp(l2_v[pl.ds(o, 16)] - m2w)
        return (s1v, s2v)

    s1v, s2v = lax.fori_loop(0, CH // 64, sbody, (zv16, zv16))
    s1w = jnp.sum(s1v)
    s2w = jnp.sum(s2v)
    lane = lax.iota(jnp.int32, 16)
    stat_v[...] = jnp.where(
        lane == 0, m1w,
        jnp.where(lane == 1, s1w,
                  jnp.where(lane == 2, m2w,
                            jnp.where(lane == 3, s2w, 0.0))))
    pltpu.sync_copy(stat_v, stats_out.at[wid])

    pltpu.sync_copy(l1_v, l1_out.at[pl.ds(base, CH)])
    pltpu.sync_copy(l2_v, l2_out.at[pl.ds(base, CH)])
    pltpu.sync_copy(idx_v, idx_out.at[pl.ds(base, CH)])


def _make_logits():
    return pl.kernel(
        _logits_body,
        out_type=[
            jax.ShapeDtypeStruct((E,), jnp.float32),
            jax.ShapeDtypeStruct((E,), jnp.float32),
            jax.ShapeDtypeStruct((E,), jnp.int32),
            jax.ShapeDtypeStruct((NW, 16), jnp.float32),
        ],
        mesh=plsc.VectorSubcoreMesh(core_axis_name="c", subcore_axis_name="s"),
        compiler_params=pltpu.CompilerParams(needs_layout_passes=False, disable_bounds_checks=True),
        scratch_types=[
            pltpu.VMEM((N,), jnp.float32),
            pltpu.VMEM((N,), jnp.float32),
            pltpu.VMEM((CH,), jnp.int32),
            pltpu.VMEM((CH,), jnp.int32),
            pltpu.VMEM((CH,), jnp.float32),
            pltpu.VMEM((CH,), jnp.float32),
            pltpu.VMEM((CH,), jnp.int32),
            pltpu.VMEM((16,), jnp.float32),
        ],
    )


def _bgather_body(xn_hbm, col_hbm, x1_out, c_v, rows0, rows1, sem0, sem1):
    wid = lax.axis_index("s") * NC + lax.axis_index("c")
    base = wid * CHH
    pltpu.sync_copy(col_hbm.at[pl.ds(base, CHH)], c_v)

    def gstart(j, buf, sem):
        go = pl.multiple_of(j * GC, 8)
        pltpu.async_copy(xn_hbm.at[c_v.at[pl.ds(go, GC)]], buf, sem)

    def gfinish(j, buf, sem):
        pltpu.make_async_copy(xn_hbm.at[c_v.at[pl.ds(0, GC)]], buf, sem).wait()
        go = pl.multiple_of(j * GC, 8)
        pltpu.sync_copy(buf, x1_out.at[pl.ds(base + go, GC)])

    gstart(0, rows0, sem0)

    def gbody(p, carry):
        j0 = 2 * p
        gstart(j0 + 1, rows1, sem1)
        gfinish(j0, rows0, sem0)
        gstart(j0 + 2, rows0, sem0)
        gfinish(j0 + 1, rows1, sem1)
        return carry

    lax.fori_loop(0, (NGH - 1) // 2, gbody, 0)
    gfinish(NGH - 1, rows0, sem0)


def _make_bgather():
    return pl.kernel(
        _bgather_body,
        out_type=jax.ShapeDtypeStruct((EH, ND), jnp.float32),
        mesh=plsc.VectorSubcoreMesh(core_axis_name="c", subcore_axis_name="s"),
        compiler_params=pltpu.CompilerParams(needs_layout_passes=False, disable_bounds_checks=True),
        scratch_types=[
            pltpu.VMEM((CHH,), jnp.int32),
            pltpu.VMEM((GC, ND), jnp.float32),
            pltpu.VMEM((GC, ND), jnp.float32),
            pltpu.SemaphoreType.DMA,
            pltpu.SemaphoreType.DMA,
        ],
    )


def _stats_body(p_ref, o_ref):
    p = p_ref[...]
    m1w = p[:, 0:1]
    s1w = p[:, 1:2]
    m2w = p[:, 2:3]
    s2w = p[:, 3:4]
    m1 = jnp.max(m1w)
    m2 = jnp.max(m2w)
    s1 = jnp.sum(s1w * jnp.exp(m1w - m1))
    s2 = jnp.sum(s2w * jnp.exp(m2w - m2))
    o_ref[0] = m1
    o_ref[1] = jnp.where(m1 < -1e29, 0.0, 1.0 / s1)
    o_ref[2] = m2
    o_ref[3] = jnp.where(m2 < -1e29, 0.0, 1.0 / s2)


_stats = pl.pallas_call(
    _stats_body,
    out_specs=pl.BlockSpec(memory_space=pltpu.SMEM),
    out_shape=jax.ShapeDtypeStruct((4,), jnp.float32),
)


def _mlp_body(x1_ref, ea_ref, l1_ref, l2_ref, st_ref,
              w1n_ref, w1e_ref, b1_ref, g1_ref, be1_ref,
              w2_ref, b2_ref, g2_ref, be2_ref, f_ref):
    av = (jnp.exp(l1_ref[...] - st_ref[0]) * st_ref[1]
          + jnp.exp(l2_ref[...] - st_ref[2]) * st_ref[3])
    a = av.reshape(BE, 1)
    x1 = x1_ref[...]
    ea = ea_ref[...]
    h = (jnp.dot(x1 * a, w1n_ref[...], preferred_element_type=jnp.float32)
         + jnp.dot(ea * a, w1e_ref[...], preferred_element_type=jnp.float32)
         + b1_ref[...])
    h = _ln_relu(h, g1_ref[...], be1_ref[...])
    h = jnp.dot(h, w2_ref[...], preferred_element_type=jnp.float32) + b2_ref[...]
    f_ref[...] = _ln_relu(h, g2_ref[...], be2_ref[...])


_mlp = pl.pallas_call(
    _mlp_body,
    grid=(pl.cdiv(EH, BE),),
    in_specs=[
        pl.BlockSpec((BE, ND), lambda i: (i, 0)),
        pl.BlockSpec((BE, DE), lambda i: (i, 0)),
        pl.BlockSpec((BE,), lambda i: (i,)),
        pl.BlockSpec((BE,), lambda i: (i,)),
        pl.BlockSpec(memory_space=pltpu.SMEM),
        pl.BlockSpec((ND, ND), lambda i: (0, 0)),
        pl.BlockSpec((DE, ND), lambda i: (0, 0)),
        pl.BlockSpec((1, ND), lambda i: (0, 0)),
        pl.BlockSpec((1, ND), lambda i: (0, 0)),
        pl.BlockSpec((1, ND), lambda i: (0, 0)),
        pl.BlockSpec((ND, ND), lambda i: (0, 0)),
        pl.BlockSpec((1, ND), lambda i: (0, 0)),
        pl.BlockSpec((1, ND), lambda i: (0, 0)),
        pl.BlockSpec((1, ND), lambda i: (0, 0)),
    ],
    out_specs=pl.BlockSpec((BE, ND), lambda i: (i, 0)),
    out_shape=jax.ShapeDtypeStruct((EH, ND), jnp.float32),
)


def _scatter_body(f_hbm, idx_hbm, agg_out, f_v0, f_v1, idx_v0, idx_v1,
                  sidx_v, sidx_t, acc_sh, sem0, sem1):
    cid = lax.axis_index("c")
    sid = lax.axis_index("s")

    zv = jnp.zeros((16,), jnp.float32)

    def zb(i, carry):
        for k in range(ND // 16):
            f_v0[i, pl.ds(k * 16, 16)] = zv
        return carry

    lax.fori_loop(0, DC, zb, 0)
    for q in range(ZR):
        pltpu.sync_copy(f_v0, acc_sh.at[pl.ds((sid * ZR + q) * DC, DC)])
    plsc.subcore_barrier()

    tb = sid * TCH

    def start(j, fv, iv, sem):
        off = pl.multiple_of(tb + j * DC, 8)
        pltpu.async_copy(idx_hbm.at[pl.ds(off, DC)], iv, sem)
        pltpu.async_copy(f_hbm.at[pl.ds(off, DC)], fv, sem)

    def process(fv, iv, sem):
        pltpu.make_async_copy(idx_hbm.at[pl.ds(0, DC)], iv, sem).wait()
        pltpu.make_async_copy(f_hbm.at[pl.ds(0, DC)], fv, sem).wait()

        def ib(k, c2):
            o = pl.multiple_of(k * 16, 16)
            v = iv[pl.ds(o, 16)]
            lo = v - cid * N
            ok = jnp.logical_and(lo >= 0, lo < N)
            sidx_v[pl.ds(o, 16)] = jnp.where(ok, lo, N)
            return c2

        lax.fori_loop(0, DC // 16, ib, 0)
        pltpu.sync_copy(fv, acc_sh.at[sidx_v], add=True)

    start(0, f_v0, idx_v0, sem0)

    def body(p, carry):
        j0 = 2 * p
        start(j0 + 1, f_v1, idx_v1, sem1)
        process(f_v0, idx_v0, sem0)

        @pl.when(p < NDC // 2 - 1)
        def _next():
            start(j0 + 2, f_v0, idx_v0, sem0)

        process(f_v1, idx_v1, sem1)
        return carry

    lax.fori_loop(0, NDC // 2, body, 0)
    toff = pl.multiple_of(tb + NDC * DC, 8)
    pltpu.sync_copy(idx_hbm.at[pl.ds(toff, DCT)], idx_v0.at[pl.ds(0, DCT)])
    pltpu.sync_copy(f_hbm.at[pl.ds(toff, DCT)], f_v0.at[pl.ds(0, DCT)])

    def tib(k, c2):
        o = pl.multiple_of(k * 16, 16)
        v = idx_v0[pl.ds(o, 16)]
        lo = v - cid * N
        ok = jnp.logical_and(lo >= 0, lo < N)
        sidx_t[pl.ds(o, 16)] = jnp.where(ok, lo, N)
        return c2

    lax.fori_loop(0, DCT // 16, tib, 0)
    pltpu.sync_copy(f_v0.at[pl.ds(0, DCT)], acc_sh.at[sidx_t], add=True)
    plsc.subcore_barrier()
    wb = sid * WCH
    pltpu.sync_copy(acc_sh.at[pl.ds(wb, WCH)],
                    agg_out.at[pl.ds(cid * N + wb, WCH)])

    @pl.when(sid == NS - 1)
    def _tail():
        pltpu.sync_copy(acc_sh.at[pl.ds(NS * WCH, N - NS * WCH)],
                        agg_out.at[pl.ds(cid * N + NS * WCH, N - NS * WCH)])


def _make_scatter():
    return pl.kernel(
        _scatter_body,
        out_type=jax.ShapeDtypeStruct((2 * N, ND), jnp.float32),
        mesh=plsc.VectorSubcoreMesh(core_axis_name="c", subcore_axis_name="s"),
        compiler_params=pltpu.CompilerParams(needs_layout_passes=False, disable_bounds_checks=True),
        scratch_types=[
            pltpu.VMEM((DC, ND), jnp.float32),
            pltpu.VMEM((DC, ND), jnp.float32),
            pltpu.VMEM((DC,), jnp.int32),
            pltpu.VMEM((DC,), jnp.int32),
            pltpu.VMEM((DC,), jnp.int32),
            pltpu.VMEM((DCT,), jnp.int32),
            pltpu.VMEM_SHARED((TOT, ND), jnp.float32),
            pltpu.SemaphoreType.DMA,
            pltpu.SemaphoreType.DMA,
        ],
    )


def _upd_body(aT0_ref, aT1_ref, aD0_ref, aD1_ref, x_ref, s_ref,
              wnT_ref, wnD_ref, bn_ref, gn_ref, ben_ref, o_ref):
    aT = aT0_ref[...] + aT1_ref[...]
    aD = aD0_ref[...] + aD1_ref[...]
    h = (jnp.dot(aT, wnT_ref[...], preferred_element_type=jnp.float32)
         + jnp.dot(aD, wnD_ref[...], preferred_element_type=jnp.float32)
         + bn_ref[...])
    upd = _ln_relu(h, gn_ref[...], ben_ref[...]) + s_ref[...]
    o_ref[...] = jnp.concatenate([upd, x_ref[:, ND:]], axis=1)


_upd = pl.pallas_call(
    _upd_body,
    grid=(N // BN,),
    in_specs=[
        pl.BlockSpec((BN, ND), lambda i: (i, 0)),
        pl.BlockSpec((BN, ND), lambda i: (i, 0)),
        pl.BlockSpec((BN, ND), lambda i: (i + N // BN, 0)),
        pl.BlockSpec((BN, ND), lambda i: (i + N // BN, 0)),
        pl.BlockSpec((BN, ND + DE), lambda i: (i, 0)),
        pl.BlockSpec((BN, ND), lambda i: (i, 0)),
        pl.BlockSpec((ND, ND), lambda i: (0, 0)),
        pl.BlockSpec((ND, ND), lambda i: (0, 0)),
        pl.BlockSpec((1, ND), lambda i: (0, 0)),
        pl.BlockSpec((1, ND), lambda i: (0, 0)),
        pl.BlockSpec((1, ND), lambda i: (0, 0)),
    ],
    out_specs=pl.BlockSpec((BN, ND + DE), lambda i: (i, 0)),
    out_shape=jax.ShapeDtypeStruct((N, ND + DE), jnp.float32),
)


def kernel(x, edge_index, edge_attr, att_W, att_b, W1, b1, g1, be1,
           W2, b2, g2, be2, Wn, bn, gn, ben, Ws, bs, gs, bes):
    x = x.astype(jnp.float32)
    ei = edge_index.astype(jnp.int32)
    row = ei[0]
    col = ei[1]

    xn_c, pc, pr, S = _prep(
        x, att_W[:ND], att_W[ND:], att_b, Ws,
        bs.reshape(1, ND), gs.reshape(1, ND), bes.reshape(1, ND))

    l1, l2, eidx, stp = _make_logits()(row, col, pc.reshape(N), pr.reshape(N))
    st = _stats(stp)

    bg = _make_bgather()
    x1a = bg(xn_c, col[:EH])
    x1b = bg(xn_c, col[EH:])

    w_args = (W1[:ND], W1[ND:], b1.reshape(1, ND), g1.reshape(1, ND),
              be1.reshape(1, ND), W2, b2.reshape(1, ND), g2.reshape(1, ND),
              be2.reshape(1, ND))
    fa = _mlp(x1a, edge_attr[:EH], l1[:EH], l2[:EH], st, *w_args)
    fb = _mlp(x1b, edge_attr[EH:], l1[EH:], l2[EH:], st, *w_args)

    sc = _make_scatter()
    agg_a = sc(fa, eidx[:EH])
    agg_b = sc(fb, eidx[EH:])

    return _upd(agg_a, agg_b, agg_a, agg_b, x, S, Wn[:ND], Wn[ND:],
                bn.reshape(1, ND), gn.reshape(1, ND), ben.reshape(1, ND))


# speedup vs baseline: 1.0204x; 1.0204x over previous
import jax
import jax.numpy as jnp
from jax import lax
from jax.experimental import pallas as pl
from jax.experimental.pallas import tpu as pltpu
from jax.experimental.pallas import tpu_sc as plsc

N = 10000
E = 320000
EA = 192000
EB = E - EA
ND = 128
DE = 16

NC = 2
NS = 16
NW = NC * NS
CH = E // NW
GC = 200
DC = 160
ZR = 4
TOT = NS * ZR * DC
WCH = 624
NEG = -1e30

BA = 1000
BE = 2048
BN = 1000


def _ln_relu(h, g, b):
    mu = jnp.mean(h, axis=-1, keepdims=True)
    d = h - mu
    var = jnp.mean(d * d, axis=-1, keepdims=True)
    return jnp.maximum(d * lax.rsqrt(var + 1e-5) * g + b, 0.0)


def _prep_body(x_ref, wa_ref, wb_ref, ab_ref, ws_ref, bs_ref, gs_ref, bes_ref,
               xn_ref, pc_ref, pr_ref, s_ref):
    xn = x_ref[:, :ND]
    xn_ref[...] = xn
    pc_ref[...] = jnp.dot(xn, wa_ref[...], preferred_element_type=jnp.float32) + ab_ref[0]
    pr_ref[...] = jnp.dot(xn, wb_ref[...], preferred_element_type=jnp.float32)
    h = jnp.dot(xn, ws_ref[...], preferred_element_type=jnp.float32) + bs_ref[...]
    s_ref[...] = _ln_relu(h, gs_ref[...], bes_ref[...])


_prep = pl.pallas_call(
    _prep_body,
    grid=(N // BA,),
    in_specs=[
        pl.BlockSpec((BA, ND + DE), lambda i: (i, 0)),
        pl.BlockSpec((ND, 1), lambda i: (0, 0)),
        pl.BlockSpec((ND, 1), lambda i: (0, 0)),
        pl.BlockSpec(memory_space=pltpu.SMEM),
        pl.BlockSpec((ND, ND), lambda i: (0, 0)),
        pl.BlockSpec((1, ND), lambda i: (0, 0)),
        pl.BlockSpec((1, ND), lambda i: (0, 0)),
        pl.BlockSpec((1, ND), lambda i: (0, 0)),
    ],
    out_specs=[
        pl.BlockSpec((BA, ND), lambda i: (i, 0)),
        pl.BlockSpec((BA, 1), lambda i: (i, 0)),
        pl.BlockSpec((BA, 1), lambda i: (i, 0)),
        pl.BlockSpec((BA, ND), lambda i: (i, 0)),
    ],
    out_shape=[
        jax.ShapeDtypeStruct((N, ND), jnp.float32),
        jax.ShapeDtypeStruct((N, 1), jnp.float32),
        jax.ShapeDtypeStruct((N, 1), jnp.float32),
        jax.ShapeDtypeStruct((N, ND), jnp.float32),
    ],
)


def _logits_body(row_hbm, col_hbm, pc_hbm, pr_hbm,
                 l1_out, l2_out, idx_out, stats_out,
                 pc_v, pr_v, r_v, c_v, l1_v, l2_v, idx_v, stat_v):
    wid = lax.axis_index("s") * NC + lax.axis_index("c")
    base = wid * CH
    pltpu.sync_copy(pc_hbm, pc_v)
    pltpu.sync_copy(pr_hbm, pr_v)
    pltpu.sync_copy(row_hbm.at[pl.ds(base, CH)], r_v)
    pltpu.sync_copy(col_hbm.at[pl.ds(base, CH)], c_v)

    negv = jnp.full((16,), NEG, jnp.float32)

    def body(i, carry):
        m1v, m2v = carry
        for k in range(4):
            o = pl.multiple_of(i * 64 + k * 16, 16)
            r = r_v[pl.ds(o, 16)]
            c = c_v[pl.ds(o, 16)]
            lc = plsc.load_gather(pc_v, [c])
            lr = plsc.load_gather(pr_v, [r])
            s = lc + lr
            l = jnp.where(s >= 0.0, s, s * 0.01)
            m1 = r < c
            m2 = r > c
            l1 = jnp.where(m1, l, NEG)
            l2 = jnp.where(m2, l, NEG)
            l1_v[pl.ds(o, 16)] = l1
            l2_v[pl.ds(o, 16)] = l2
            idx_v[pl.ds(o, 16)] = jnp.where(m1, r + N, jnp.where(m2, r, 2 * N))
            m1v = jnp.maximum(m1v, l1)
            m2v = jnp.maximum(m2v, l2)
        return (m1v, m2v)

    m1v, m2v = lax.fori_loop(0, CH // 64, body, (negv, negv))
    m1w = jnp.max(m1v)
    m2w = jnp.max(m2v)
    zv16 = jnp.zeros((16,), jnp.float32)

    def sbody(i, carry):
        s1v, s2v = carry
        for k in range(4):
            o = pl.multiple_of(i * 64 + k * 16, 16)
            s1v = s1v + jnp.exp(l1_v[pl.ds(o, 16)] - m1w)
            s2v = s2v + jnp.exp(l2_v[pl.ds(o, 16)] - m2w)
        return (s1v, s2v)

    s1v, s2v = lax.fori_loop(0, CH // 64, sbody, (zv16, zv16))
    s1w = jnp.sum(s1v)
    s2w = jnp.sum(s2v)
    lane = lax.iota(jnp.int32, 16)
    stat_v[...] = jnp.where(
        lane == 0, m1w,
        jnp.where(lane == 1, s1w,
                  jnp.where(lane == 2, m2w,
                            jnp.where(lane == 3, s2w, 0.0))))
    pltpu.sync_copy(stat_v, stats_out.at[wid])

    pltpu.sync_copy(l1_v, l1_out.at[pl.ds(base, CH)])
    pltpu.sync_copy(l2_v, l2_out.at[pl.ds(base, CH)])
    pltpu.sync_copy(idx_v, idx_out.at[pl.ds(base, CH)])


def _make_logits():
    return pl.kernel(
        _logits_body,
        out_type=[
            jax.ShapeDtypeStruct((E,), jnp.float32),
            jax.ShapeDtypeStruct((E,), jnp.float32),
            jax.ShapeDtypeStruct((E,), jnp.int32),
            jax.ShapeDtypeStruct((NW, 16), jnp.float32),
        ],
        mesh=plsc.VectorSubcoreMesh(core_axis_name="c", subcore_axis_name="s"),
        compiler_params=pltpu.CompilerParams(needs_layout_passes=False, disable_bounds_checks=True),
        scratch_types=[
            pltpu.VMEM((N,), jnp.float32),
            pltpu.VMEM((N,), jnp.float32),
            pltpu.VMEM((CH,), jnp.int32),
            pltpu.VMEM((CH,), jnp.int32),
            pltpu.VMEM((CH,), jnp.float32),
            pltpu.VMEM((CH,), jnp.float32),
            pltpu.VMEM((CH,), jnp.int32),
            pltpu.VMEM((16,), jnp.float32),
        ],
    )


def _make_bgather(m):
    chw = m // NW
    ng = chw // GC

    def _bgather_body(xn_hbm, col_hbm, x1_out, c_v, rows0, rows1, sem0, sem1):
        wid = lax.axis_index("s") * NC + lax.axis_index("c")
        base = wid * chw
        pltpu.sync_copy(col_hbm.at[pl.ds(base, chw)], c_v)

        def gstart(j, buf, sem):
            go = pl.multiple_of(j * GC, 8)
            pltpu.async_copy(xn_hbm.at[c_v.at[pl.ds(go, GC)]], buf, sem)

        def gfinish(j, buf, sem):
            pltpu.make_async_copy(xn_hbm.at[c_v.at[pl.ds(0, GC)]], buf, sem).wait()
            go = pl.multiple_of(j * GC, 8)
            pltpu.sync_copy(buf, x1_out.at[pl.ds(base + go, GC)])

        gstart(0, rows0, sem0)

        def gbody(p, carry):
            j0 = 2 * p
            gstart(j0 + 1, rows1, sem1)
            gfinish(j0, rows0, sem0)
            gstart(j0 + 2, rows0, sem0)
            gfinish(j0 + 1, rows1, sem1)
            return carry

        lax.fori_loop(0, (ng - 1) // 2, gbody, 0)
        if ng % 2 == 1:
            gfinish(ng - 1, rows0, sem0)
        else:
            gstart(ng - 1, rows1, sem1)
            gfinish(ng - 2, rows0, sem0)
            gfinish(ng - 1, rows1, sem1)

    return pl.kernel(
        _bgather_body,
        out_type=jax.ShapeDtypeStruct((m, ND), jnp.float32),
        mesh=plsc.VectorSubcoreMesh(core_axis_name="c", subcore_axis_name="s"),
        compiler_params=pltpu.CompilerParams(needs_layout_passes=False, disable_bounds_checks=True),
        scratch_types=[
            pltpu.VMEM((chw,), jnp.int32),
            pltpu.VMEM((GC, ND), jnp.float32),
            pltpu.VMEM((GC, ND), jnp.float32),
            pltpu.SemaphoreType.DMA,
            pltpu.SemaphoreType.DMA,
        ],
    )


def _stats_body(p_ref, o_ref):
    p = p_ref[...]
    m1w = p[:, 0:1]
    s1w = p[:, 1:2]
    m2w = p[:, 2:3]
    s2w = p[:, 3:4]
    m1 = jnp.max(m1w)
    m2 = jnp.max(m2w)
    s1 = jnp.sum(s1w * jnp.exp(m1w - m1))
    s2 = jnp.sum(s2w * jnp.exp(m2w - m2))
    o_ref[0] = m1
    o_ref[1] = jnp.where(m1 < -1e29, 0.0, 1.0 / s1)
    o_ref[2] = m2
    o_ref[3] = jnp.where(m2 < -1e29, 0.0, 1.0 / s2)


_stats = pl.pallas_call(
    _stats_body,
    out_specs=pl.BlockSpec(memory_space=pltpu.SMEM),
    out_shape=jax.ShapeDtypeStruct((4,), jnp.float32),
)


def _mlp_body(x1_ref, ea_ref, l1_ref, l2_ref, st_ref,
              w1n_ref, w1e_ref, b1_ref, g1_ref, be1_ref,
              w2_ref, b2_ref, g2_ref, be2_ref, f_ref):
    av = (jnp.exp(l1_ref[...] - st_ref[0]) * st_ref[1]
          + jnp.exp(l2_ref[...] - st_ref[2]) * st_ref[3])
    a = av.reshape(BE, 1)
    x1 = x1_ref[...]
    ea = ea_ref[...]
    h = (jnp.dot(x1 * a, w1n_ref[...], preferred_element_type=jnp.float32)
         + jnp.dot(ea * a, w1e_ref[...], preferred_element_type=jnp.float32)
         + b1_ref[...])
    h = _ln_relu(h, g1_ref[...], be1_ref[...])
    h = jnp.dot(h, w2_ref[...], preferred_element_type=jnp.float32) + b2_ref[...]
    f_ref[...] = _ln_relu(h, g2_ref[...], be2_ref[...])


def _make_mlp(m):
    return pl.pallas_call(
        _mlp_body,
        grid=(pl.cdiv(m, BE),),
        in_specs=[
        pl.BlockSpec((BE, ND), lambda i: (i, 0)),
        pl.BlockSpec((BE, DE), lambda i: (i, 0)),
        pl.BlockSpec((BE,), lambda i: (i,)),
        pl.BlockSpec((BE,), lambda i: (i,)),
        pl.BlockSpec(memory_space=pltpu.SMEM),
        pl.BlockSpec((ND, ND), lambda i: (0, 0)),
        pl.BlockSpec((DE, ND), lambda i: (0, 0)),
        pl.BlockSpec((1, ND), lambda i: (0, 0)),
        pl.BlockSpec((1, ND), lambda i: (0, 0)),
        pl.BlockSpec((1, ND), lambda i: (0, 0)),
        pl.BlockSpec((ND, ND), lambda i: (0, 0)),
        pl.BlockSpec((1, ND), lambda i: (0, 0)),
        pl.BlockSpec((1, ND), lambda i: (0, 0)),
        pl.BlockSpec((1, ND), lambda i: (0, 0)),
        ],
        out_specs=pl.BlockSpec((BE, ND), lambda i: (i, 0)),
        out_shape=jax.ShapeDtypeStruct((m, ND), jnp.float32),
    )


def _make_scatter(m):
    tch = m // NS
    ndc = tch // DC

    def _scatter_body(f_hbm, idx_hbm, agg_out, f_v0, f_v1, idx_v0, idx_v1,
                      sidx_v, acc_sh, sem0, sem1):
        cid = lax.axis_index("c")
        sid = lax.axis_index("s")

        zv = jnp.zeros((16,), jnp.float32)

        def zb(i, carry):
            for k in range(ND // 16):
                f_v0[i, pl.ds(k * 16, 16)] = zv
            return carry

        lax.fori_loop(0, DC, zb, 0)
        for q in range(ZR):
            pltpu.sync_copy(f_v0, acc_sh.at[pl.ds((sid * ZR + q) * DC, DC)])
        plsc.subcore_barrier()

        tb = sid * tch

        def start(j, fv, iv, sem):
            off = pl.multiple_of(tb + j * DC, 8)
            pltpu.async_copy(idx_hbm.at[pl.ds(off, DC)], iv, sem)
            pltpu.async_copy(f_hbm.at[pl.ds(off, DC)], fv, sem)

        def process(fv, iv, sem):
            pltpu.make_async_copy(idx_hbm.at[pl.ds(0, DC)], iv, sem).wait()
            pltpu.make_async_copy(f_hbm.at[pl.ds(0, DC)], fv, sem).wait()

            def ib(k, c2):
                o = pl.multiple_of(k * 16, 16)
                v = iv[pl.ds(o, 16)]
                lo = v - cid * N
                ok = jnp.logical_and(lo >= 0, lo < N)
                sidx_v[pl.ds(o, 16)] = jnp.where(ok, lo, N)
                return c2

            lax.fori_loop(0, DC // 16, ib, 0)
            pltpu.sync_copy(fv, acc_sh.at[sidx_v], add=True)

        start(0, f_v0, idx_v0, sem0)

        def body(p, carry):
            j0 = 2 * p
            start(j0 + 1, f_v1, idx_v1, sem1)
            process(f_v0, idx_v0, sem0)

            @pl.when(p < ndc // 2 - 1)
            def _next():
                start(j0 + 2, f_v0, idx_v0, sem0)

            process(f_v1, idx_v1, sem1)
            return carry

        lax.fori_loop(0, ndc // 2, body, 0)
        if ndc % 2 == 1:
            start(ndc - 1, f_v0, idx_v0, sem0)
            process(f_v0, idx_v0, sem0)
        plsc.subcore_barrier()
        wb = sid * WCH
        pltpu.sync_copy(acc_sh.at[pl.ds(wb, WCH)],
                        agg_out.at[pl.ds(cid * N + wb, WCH)])

        @pl.when(sid == NS - 1)
        def _tail():
            pltpu.sync_copy(acc_sh.at[pl.ds(NS * WCH, N - NS * WCH)],
                            agg_out.at[pl.ds(cid * N + NS * WCH, N - NS * WCH)])

    return pl.kernel(
        _scatter_body,
        out_type=jax.ShapeDtypeStruct((2 * N, ND), jnp.float32),
        mesh=plsc.VectorSubcoreMesh(core_axis_name="c", subcore_axis_name="s"),
        compiler_params=pltpu.CompilerParams(needs_layout_passes=False, disable_bounds_checks=True),
        scratch_types=[
            pltpu.VMEM((DC, ND), jnp.float32),
            pltpu.VMEM((DC, ND), jnp.float32),
            pltpu.VMEM((DC,), jnp.int32),
            pltpu.VMEM((DC,), jnp.int32),
            pltpu.VMEM((DC,), jnp.int32),
            pltpu.VMEM_SHARED((TOT, ND), jnp.float32),
            pltpu.SemaphoreType.DMA,
            pltpu.SemaphoreType.DMA,
        ],
    )


def _upd_body(aT0_ref, aT1_ref, aD0_ref, aD1_ref, x_ref, s_ref,
              wnT_ref, wnD_ref, bn_ref, gn_ref, ben_ref, o_ref):
    aT = aT0_ref[...] + aT1_ref[...]
    aD = aD0_ref[...] + aD1_ref[...]
    h = (jnp.dot(aT, wnT_ref[...], preferred_element_type=jnp.float32)
         + jnp.dot(aD, wnD_ref[...], preferred_element_type=jnp.float32)
         + bn_ref[...])
    upd = _ln_relu(h, gn_ref[...], ben_ref[...]) + s_ref[...]
    o_ref[...] = jnp.concatenate([upd, x_ref[:, ND:]], axis=1)


_upd = pl.pallas_call(
    _upd_body,
    grid=(N // BN,),
    in_specs=[
        pl.BlockSpec((BN, ND), lambda i: (i, 0)),
        pl.BlockSpec((BN, ND), lambda i: (i, 0)),
        pl.BlockSpec((BN, ND), lambda i: (i + N // BN, 0)),
        pl.BlockSpec((BN, ND), lambda i: (i + N // BN, 0)),
        pl.BlockSpec((BN, ND + DE), lambda i: (i, 0)),
        pl.BlockSpec((BN, ND), lambda i: (i, 0)),
        pl.BlockSpec((ND, ND), lambda i: (0, 0)),
        pl.BlockSpec((ND, ND), lambda i: (0, 0)),
        pl.BlockSpec((1, ND), lambda i: (0, 0)),
        pl.BlockSpec((1, ND), lambda i: (0, 0)),
        pl.BlockSpec((1, ND), lambda i: (0, 0)),
    ],
    out_specs=pl.BlockSpec((BN, ND + DE), lambda i: (i, 0)),
    out_shape=jax.ShapeDtypeStruct((N, ND + DE), jnp.float32),
)


def kernel(x, edge_index, edge_attr, att_W, att_b, W1, b1, g1, be1,
           W2, b2, g2, be2, Wn, bn, gn, ben, Ws, bs, gs, bes):
    x = x.astype(jnp.float32)
    ei = edge_index.astype(jnp.int32)
    row = ei[0]
    col = ei[1]

    xn_c, pc, pr, S = _prep(
        x, att_W[:ND], att_W[ND:], att_b, Ws,
        bs.reshape(1, ND), gs.reshape(1, ND), bes.reshape(1, ND))

    l1, l2, eidx, stp = _make_logits()(row, col, pc.reshape(N), pr.reshape(N))
    st = _stats(stp)

    x1a = _make_bgather(EA)(xn_c, col[:EA])
    x1b = _make_bgather(EB)(xn_c, col[EA:])

    w_args = (W1[:ND], W1[ND:], b1.reshape(1, ND), g1.reshape(1, ND),
              be1.reshape(1, ND), W2, b2.reshape(1, ND), g2.reshape(1, ND),
              be2.reshape(1, ND))
    fa = _make_mlp(EA)(x1a, edge_attr[:EA], l1[:EA], l2[:EA], st, *w_args)
    fb = _make_mlp(EB)(x1b, edge_attr[EA:], l1[EA:], l2[EA:], st, *w_args)

    agg_a = _make_scatter(EA)(fa, eidx[:EA])
    agg_b = _make_scatter(EB)(fb, eidx[EA:])

    return _upd(agg_a, agg_b, agg_a, agg_b, x, S, Wn[:ND], Wn[ND:],
                bn.reshape(1, ND), gn.reshape(1, ND), ben.reshape(1, ND))
